# baseline (speedup 1.0000x reference)
import functools
import jax
import jax.numpy as jnp
from jax import lax
from jax.experimental import pallas as pl
from jax.experimental.pallas import tpu as pltpu
from jax.experimental.pallas import tpu_sc as plsc

_D = 4096
_SEQ = 8192
_BATCH = 4
_NE = 32
_TAIL = 2048
_HEAD = _SEQ - _TAIL
_SBLK = 512
_NW = 32
_WPB = _NW // _BATCH
_RPW = _TAIL // _WPB
_G = 8
_NCHUNK = _RPW // _G
_NVEC = _D // 16


def _sc_pool_body(x_hbm, out_hbm, buf0, buf1, acc, sem0, sem1):
    c = lax.axis_index("c")
    s = lax.axis_index("s")
    w = s * 2 + c
    b = w // _WPB
    ch = w % _WPB
    base = b * _SEQ + _HEAD + ch * _RPW

    zero = jnp.zeros((16,), jnp.float32)
    for k in range(_NVEC):
        acc[pl.ds(k * 16, 16)] = zero

    bufs = (buf0, buf1)
    sems = (sem0, sem1)
    for i in range(2):
        pltpu.async_copy(x_hbm.at[pl.ds(base + i * _G, _G)], bufs[i], sems[i])

    def outer(g, carry):
        for i in range(2):
            cidx = g * 2 + i
            pltpu.make_async_copy(x_hbm.at[pl.ds(base, _G)], bufs[i],
                                  sems[i]).wait()

            def inner(r, c2):
                for k in range(_NVEC):
                    plsc.addupdate(acc.at[pl.ds(k * 16, 16)],
                                   bufs[i][r, pl.ds(k * 16, 16)])
                return c2

            lax.fori_loop(0, _G, inner, 0)

            @pl.when(cidx + 2 < _NCHUNK)
            def _():
                pltpu.async_copy(x_hbm.at[pl.ds(base + (cidx + 2) * _G, _G)],
                                 bufs[i], sems[i])
        return carry

    lax.fori_loop(0, _NCHUNK // 2, outer, 0)
    pltpu.sync_copy(acc, out_hbm.at[w])


def _sc_pool(xf):
    mesh = plsc.VectorSubcoreMesh(core_axis_name="c", subcore_axis_name="s")
    return pl.kernel(
        _sc_pool_body,
        out_type=jax.ShapeDtypeStruct((_NW, _D), jnp.float32),
        mesh=mesh,
        scratch_types=[
            pltpu.VMEM((_G, _D), jnp.float32),
            pltpu.VMEM((_G, _D), jnp.float32),
            pltpu.VMEM((_D,), jnp.float32),
            pltpu.SemaphoreType.DMA,
            pltpu.SemaphoreType.DMA,
        ],
    )(xf)


def _tc_kernel(x_ref, sc_ref, w1_ref, b1_ref, g1_ref, be1_ref,
               w2_ref, b2_ref, g2_ref, be2_ref,
               w3_ref, b3_ref, o_ref, acc_ref):
    b = pl.program_id(0)
    s = pl.program_id(1)
    ns = pl.num_programs(1)
    part = jnp.sum(x_ref[...], axis=1)

    @pl.when(s == 0)
    def _():
        acc_ref[pl.ds(b, 1), :] = part

    @pl.when(s > 0)
    def _():
        acc_ref[pl.ds(b, 1), :] += part

    @pl.when((b == _BATCH - 1) & (s == ns - 1))
    def _():
        pooled = (acc_ref[...] + jnp.sum(sc_ref[...], axis=1)) * (1.0 / _SEQ)

        def _ln(h, g, bb, eps=1e-5):
            m = jnp.mean(h, axis=-1, keepdims=True)
            v = jnp.mean((h - m) ** 2, axis=-1, keepdims=True)
            return (h - m) / jnp.sqrt(v + eps) * g + bb

        h = jax.lax.dot_general(pooled, w1_ref[...], (((1,), (1,)), ((), ())),
                                preferred_element_type=jnp.float32) + b1_ref[...]
        h = jax.nn.relu(_ln(h, g1_ref[...], be1_ref[...]))
        h = jax.lax.dot_general(h, w2_ref[...], (((1,), (1,)), ((), ())),
                                preferred_element_type=jnp.float32) + b2_ref[...]
        h = jax.nn.relu(_ln(h, g2_ref[...], be2_ref[...]))
        scores = jax.lax.dot_general(h, w3_ref[...], (((1,), (1,)), ((), ())),
                                     preferred_element_type=jnp.float32) + b3_ref[...]

        scaled = scores - jnp.max(scores, axis=-1, keepdims=True)
        e = jnp.exp(scaled - jnp.max(scaled, axis=-1, keepdims=True))
        probs = e / jnp.sum(e, axis=-1, keepdims=True)

        pa = probs[:, :, None]
        pb = probs[:, None, :]
        ii = jax.lax.broadcasted_iota(jnp.int32, (1, _NE, _NE), 1)
        jj = jax.lax.broadcasted_iota(jnp.int32, (1, _NE, _NE), 2)
        beats = (pb > pa) | ((pb == pa) & (jj < ii))
        nbeat = jnp.sum(beats.astype(jnp.int32), axis=-1)
        o_ref[...] = (nbeat < 8).astype(jnp.float32)


def kernel(x, W1, b1, g1, be1, W2, b2, g2, be2, W3, b3):
    xf = x.reshape(_BATCH * _SEQ, _D)
    sc_part = _sc_pool(xf).reshape(_BATCH, _WPB, _D)

    const = lambda shape: pl.BlockSpec(shape, lambda b, s: tuple(0 for _ in shape))
    return pl.pallas_call(
        _tc_kernel,
        grid=(_BATCH, _HEAD // _SBLK),
        in_specs=[pl.BlockSpec((1, _SBLK, _D), lambda b, s: (b, s, 0)),
                  const(sc_part.shape), const(W1.shape), const(b1.shape),
                  const(g1.shape), const(be1.shape), const(W2.shape),
                  const(b2.shape), const(g2.shape), const(be2.shape),
                  const(W3.shape), const(b3.shape)],
        out_specs=const((_BATCH, _NE)),
        out_shape=jax.ShapeDtypeStruct((_BATCH, _NE), jnp.float32),
        scratch_shapes=[pltpu.VMEM((_BATCH, _D), jnp.float32)],
    )(x, sc_part, W1, b1, g1, be1, W2, b2, g2, be2, W3, b3)


# speedup vs baseline: 1.9823x; 1.9823x over previous
import jax
import jax.numpy as jnp
from jax import lax
from jax.experimental import pallas as pl
from jax.experimental.pallas import tpu as pltpu
from jax.experimental.pallas import tpu_sc as plsc

_D = 4096
_SEQ = 8192
_BATCH = 4
_NE = 32
_TAIL = 2048
_HEAD = _SEQ - _TAIL
_SBLK = 512
_NW = 32
_WPB = _NW // _BATCH
_RPW = _TAIL // _WPB
_G = 8
_NCHUNK = _RPW // _G
_NVEC = _D // 16


def _sc_pool_body(x_hbm, out_hbm, buf0, buf1, acc, sem0, sem1):
    c = lax.axis_index("c")
    s = lax.axis_index("s")
    w = s * 2 + c
    b = w // _WPB
    ch = w % _WPB
    base = b * _SEQ + _HEAD + ch * _RPW

    @plsc.parallel_loop(0, _D, 16)
    def _(off):
        acc[pl.ds(off, 16)] = jnp.zeros((16,), jnp.float32)

    bufs = (buf0, buf1)
    sems = (sem0, sem1)
    for i in range(2):
        pltpu.async_copy(x_hbm.at[pl.ds(base + i * _G, _G)], bufs[i], sems[i])

    def _accum(buf):
        @plsc.parallel_loop(0, _D, 16, unroll=4)
        def _(off):
            sl = pl.ds(off, 16)
            v01 = buf[0, sl] + buf[1, sl]
            v23 = buf[2, sl] + buf[3, sl]
            v45 = buf[4, sl] + buf[5, sl]
            v67 = buf[6, sl] + buf[7, sl]
            plsc.addupdate(acc.at[sl], (v01 + v23) + (v45 + v67))

    def outer(g, carry):
        for i in range(2):
            cidx = g * 2 + i
            pltpu.make_async_copy(x_hbm.at[pl.ds(base, _G)], bufs[i],
                                  sems[i]).wait()
            _accum(bufs[i])

            @pl.when(cidx + 2 < _NCHUNK)
            def _():
                pltpu.async_copy(x_hbm.at[pl.ds(base + (cidx + 2) * _G, _G)],
                                 bufs[i], sems[i])
        return carry

    lax.fori_loop(0, _NCHUNK // 2, outer, 0)
    pltpu.sync_copy(acc, out_hbm.at[w])


def _sc_pool(xf):
    mesh = plsc.VectorSubcoreMesh(core_axis_name="c", subcore_axis_name="s")
    return pl.kernel(
        _sc_pool_body,
        out_type=jax.ShapeDtypeStruct((_NW, _D), jnp.float32),
        mesh=mesh,
        scratch_types=[
            pltpu.VMEM((_G, _D), jnp.float32),
            pltpu.VMEM((_G, _D), jnp.float32),
            pltpu.VMEM((_D,), jnp.float32),
            pltpu.SemaphoreType.DMA,
            pltpu.SemaphoreType.DMA,
        ],
    )(xf)


def _tc_pool_kernel(x_ref, o_ref):
    b = pl.program_id(0)
    s = pl.program_id(1)
    part = jnp.sum(x_ref[...], axis=1)

    @pl.when(s == 0)
    def _():
        o_ref[pl.ds(b, 1), :] = part

    @pl.when(s > 0)
    def _():
        o_ref[pl.ds(b, 1), :] += part


def _router_kernel(tp_ref, sc_ref, w1_ref, b1_ref, g1_ref, be1_ref,
                   w2_ref, b2_ref, g2_ref, be2_ref,
                   w3_ref, b3_ref, o_ref):
    pooled = (tp_ref[...] + jnp.sum(sc_ref[...], axis=1)) * (1.0 / _SEQ)

    def _ln(h, g, bb, eps=1e-5):
        m = jnp.mean(h, axis=-1, keepdims=True)
        v = jnp.mean((h - m) ** 2, axis=-1, keepdims=True)
        return (h - m) / jnp.sqrt(v + eps) * g + bb

    h = jax.lax.dot_general(pooled, w1_ref[...], (((1,), (1,)), ((), ())),
                            preferred_element_type=jnp.float32) + b1_ref[...]
    h = jax.nn.relu(_ln(h, g1_ref[...], be1_ref[...]))
    h = jax.lax.dot_general(h, w2_ref[...], (((1,), (1,)), ((), ())),
                            preferred_element_type=jnp.float32) + b2_ref[...]
    h = jax.nn.relu(_ln(h, g2_ref[...], be2_ref[...]))
    scores = jax.lax.dot_general(h, w3_ref[...], (((1,), (1,)), ((), ())),
                                 preferred_element_type=jnp.float32) + b3_ref[...]

    scaled = scores - jnp.max(scores, axis=-1, keepdims=True)
    e = jnp.exp(scaled - jnp.max(scaled, axis=-1, keepdims=True))
    probs = e / jnp.sum(e, axis=-1, keepdims=True)

    pa = probs[:, :, None]
    pb = probs[:, None, :]
    ii = lax.broadcasted_iota(jnp.int32, (1, _NE, _NE), 1)
    jj = lax.broadcasted_iota(jnp.int32, (1, _NE, _NE), 2)
    beats = (pb > pa) | ((pb == pa) & (jj < ii))
    nbeat = jnp.sum(beats.astype(jnp.int32), axis=-1)
    o_ref[...] = (nbeat < 8).astype(jnp.float32)


def kernel(x, W1, b1, g1, be1, W2, b2, g2, be2, W3, b3):
    xf = x.reshape(_BATCH * _SEQ, _D)
    sc_part = _sc_pool(xf).reshape(_BATCH, _WPB, _D)

    tc_part = pl.pallas_call(
        _tc_pool_kernel,
        grid=(_BATCH, _HEAD // _SBLK),
        in_specs=[pl.BlockSpec((1, _SBLK, _D), lambda b, s: (b, s, 0))],
        out_specs=pl.BlockSpec((_BATCH, _D), lambda b, s: (0, 0)),
        out_shape=jax.ShapeDtypeStruct((_BATCH, _D), jnp.float32),
    )(x)

    full = lambda shape: pl.BlockSpec(shape, lambda: tuple(0 for _ in shape))
    return pl.pallas_call(
        _router_kernel,
        in_specs=[full(tc_part.shape), full(sc_part.shape), full(W1.shape),
                  full(b1.shape), full(g1.shape), full(be1.shape),
                  full(W2.shape), full(b2.shape), full(g2.shape),
                  full(be2.shape), full(W3.shape), full(b3.shape)],
        out_specs=full((_BATCH, _NE)),
        out_shape=jax.ShapeDtypeStruct((_BATCH, _NE), jnp.float32),
    )(tc_part, sc_part, W1, b1, g1, be1, W2, b2, g2, be2, W3, b3)


# speedup vs baseline: 1.9831x; 1.0004x over previous
import jax
import jax.numpy as jnp
from jax import lax
from jax.experimental import pallas as pl
from jax.experimental.pallas import tpu as pltpu
from jax.experimental.pallas import tpu_sc as plsc

_D = 4096
_SEQ = 8192
_BATCH = 4
_NE = 32
_TAIL = 2048
_HEAD = _SEQ - _TAIL
_SBLK = 512
_NW = 32
_WPB = _NW // _BATCH
_RPW = _TAIL // _WPB
_G = 8
_NCHUNK = _RPW // _G
_NVEC = _D // 16


def _sc_pool_body(x_hbm, out_hbm, buf0, buf1, acc, sem0, sem1):
    c = lax.axis_index("c")
    s = lax.axis_index("s")
    w = s * 2 + c
    b = w // _WPB
    ch = w % _WPB
    base = b * _SEQ + _HEAD + ch * _RPW

    @plsc.parallel_loop(0, _D, 16)
    def _(off):
        acc[pl.ds(off, 16)] = jnp.zeros((16,), jnp.float32)

    bufs = (buf0, buf1)
    sems = (sem0, sem1)
    for i in range(2):
        pltpu.async_copy(x_hbm.at[pl.ds(base + i * _G, _G)], bufs[i], sems[i])

    def _accum(buf):
        @plsc.parallel_loop(0, _D, 16, unroll=4)
        def _(off):
            sl = pl.ds(off, 16)
            v01 = buf[0, sl] + buf[1, sl]
            v23 = buf[2, sl] + buf[3, sl]
            v45 = buf[4, sl] + buf[5, sl]
            v67 = buf[6, sl] + buf[7, sl]
            plsc.addupdate(acc.at[sl], (v01 + v23) + (v45 + v67))

    def outer(g, carry):
        for i in range(2):
            cidx = g * 2 + i
            pltpu.make_async_copy(x_hbm.at[pl.ds(base, _G)], bufs[i],
                                  sems[i]).wait()
            _accum(bufs[i])

            @pl.when(cidx + 2 < _NCHUNK)
            def _():
                pltpu.async_copy(x_hbm.at[pl.ds(base + (cidx + 2) * _G, _G)],
                                 bufs[i], sems[i])
        return carry

    lax.fori_loop(0, _NCHUNK // 2, outer, 0)
    pltpu.sync_copy(acc, out_hbm.at[w])


def _sc_pool(xf):
    mesh = plsc.VectorSubcoreMesh(core_axis_name="c", subcore_axis_name="s")
    return pl.kernel(
        _sc_pool_body,
        out_type=jax.ShapeDtypeStruct((_NW, _D), jnp.float32),
        mesh=mesh,
        cost_estimate=pl.CostEstimate(
            flops=_BATCH * _TAIL * _D,
            transcendentals=0,
            bytes_accessed=_BATCH * _TAIL * _D * 4,
        ),
        scratch_types=[
            pltpu.VMEM((_G, _D), jnp.float32),
            pltpu.VMEM((_G, _D), jnp.float32),
            pltpu.VMEM((_D,), jnp.float32),
            pltpu.SemaphoreType.DMA,
            pltpu.SemaphoreType.DMA,
        ],
    )(xf)


def _tc_pool_kernel(x_ref, o_ref):
    b = pl.program_id(0)
    s = pl.program_id(1)
    part = jnp.sum(x_ref[...], axis=1)

    @pl.when(s == 0)
    def _():
        o_ref[pl.ds(b, 1), :] = part

    @pl.when(s > 0)
    def _():
        o_ref[pl.ds(b, 1), :] += part


def _router_kernel(tp_ref, sc_ref, w1_ref, b1_ref, g1_ref, be1_ref,
                   w2_ref, b2_ref, g2_ref, be2_ref,
                   w3_ref, b3_ref, o_ref):
    pooled = (tp_ref[...] + jnp.sum(sc_ref[...], axis=1)) * (1.0 / _SEQ)

    def _ln(h, g, bb, eps=1e-5):
        m = jnp.mean(h, axis=-1, keepdims=True)
        v = jnp.mean((h - m) ** 2, axis=-1, keepdims=True)
        return (h - m) / jnp.sqrt(v + eps) * g + bb

    h = jax.lax.dot_general(pooled, w1_ref[...], (((1,), (1,)), ((), ())),
                            preferred_element_type=jnp.float32) + b1_ref[...]
    h = jax.nn.relu(_ln(h, g1_ref[...], be1_ref[...]))
    h = jax.lax.dot_general(h, w2_ref[...], (((1,), (1,)), ((), ())),
                            preferred_element_type=jnp.float32) + b2_ref[...]
    h = jax.nn.relu(_ln(h, g2_ref[...], be2_ref[...]))
    scores = jax.lax.dot_general(h, w3_ref[...], (((1,), (1,)), ((), ())),
                                 preferred_element_type=jnp.float32) + b3_ref[...]

    scaled = scores - jnp.max(scores, axis=-1, keepdims=True)
    e = jnp.exp(scaled - jnp.max(scaled, axis=-1, keepdims=True))
    probs = e / jnp.sum(e, axis=-1, keepdims=True)

    pa = probs[:, :, None]
    pb = probs[:, None, :]
    ii = lax.broadcasted_iota(jnp.int32, (1, _NE, _NE), 1)
    jj = lax.broadcasted_iota(jnp.int32, (1, _NE, _NE), 2)
    beats = (pb > pa) | ((pb == pa) & (jj < ii))
    nbeat = jnp.sum(beats.astype(jnp.int32), axis=-1)
    o_ref[...] = (nbeat < 8).astype(jnp.float32)


def kernel(x, W1, b1, g1, be1, W2, b2, g2, be2, W3, b3):
    xf = x.reshape(_BATCH * _SEQ, _D)

    tc_part = pl.pallas_call(
        _tc_pool_kernel,
        grid=(_BATCH, _HEAD // _SBLK),
        in_specs=[pl.BlockSpec((1, _SBLK, _D), lambda b, s: (b, s, 0))],
        out_specs=pl.BlockSpec((_BATCH, _D), lambda b, s: (0, 0)),
        out_shape=jax.ShapeDtypeStruct((_BATCH, _D), jnp.float32),
    )(x)

    sc_part = _sc_pool(xf).reshape(_BATCH, _WPB, _D)

    full = lambda shape: pl.BlockSpec(shape, lambda: tuple(0 for _ in shape))
    return pl.pallas_call(
        _router_kernel,
        in_specs=[full(tc_part.shape), full(sc_part.shape), full(W1.shape),
                  full(b1.shape), full(g1.shape), full(be1.shape),
                  full(W2.shape), full(b2.shape), full(g2.shape),
                  full(be2.shape), full(W3.shape), full(b3.shape)],
        out_specs=full((_BATCH, _NE)),
        out_shape=jax.ShapeDtypeStruct((_BATCH, _NE), jnp.float32),
    )(tc_part, sc_part, W1, b1, g1, be1, W2, b2, g2, be2, W3, b3)


# speedup vs baseline: 1.9996x; 1.0083x over previous
import jax
import jax.numpy as jnp
from jax import lax
from jax.experimental import pallas as pl
from jax.experimental.pallas import tpu as pltpu
from jax.experimental.pallas import tpu_sc as plsc

_D = 4096
_SEQ = 8192
_BATCH = 4
_NE = 32
_TAIL = 1024
_HEAD = _SEQ - _TAIL
_SBLK = 512
_NW = 32
_WPB = _NW // _BATCH
_RPW = _TAIL // _WPB
_G = 8
_NCHUNK = _RPW // _G
_NVEC = _D // 16


def _sc_pool_body(x_hbm, out_hbm, buf0, buf1, acc, sem0, sem1):
    c = lax.axis_index("c")
    s = lax.axis_index("s")
    w = s * 2 + c
    b = w // _WPB
    ch = w % _WPB
    base = b * _SEQ + _HEAD + ch * _RPW

    @plsc.parallel_loop(0, _D, 16)
    def _(off):
        acc[pl.ds(off, 16)] = jnp.zeros((16,), jnp.float32)

    bufs = (buf0, buf1)
    sems = (sem0, sem1)
    for i in range(2):
        pltpu.async_copy(x_hbm.at[pl.ds(base + i * _G, _G)], bufs[i], sems[i])

    def _accum(buf):
        @plsc.parallel_loop(0, _D, 16, unroll=4)
        def _(off):
            sl = pl.ds(off, 16)
            v01 = buf[0, sl] + buf[1, sl]
            v23 = buf[2, sl] + buf[3, sl]
            v45 = buf[4, sl] + buf[5, sl]
            v67 = buf[6, sl] + buf[7, sl]
            plsc.addupdate(acc.at[sl], (v01 + v23) + (v45 + v67))

    def outer(g, carry):
        for i in range(2):
            cidx = g * 2 + i
            pltpu.make_async_copy(x_hbm.at[pl.ds(base, _G)], bufs[i],
                                  sems[i]).wait()
            _accum(bufs[i])

            @pl.when(cidx + 2 < _NCHUNK)
            def _():
                pltpu.async_copy(x_hbm.at[pl.ds(base + (cidx + 2) * _G, _G)],
                                 bufs[i], sems[i])
        return carry

    lax.fori_loop(0, _NCHUNK // 2, outer, 0)
    pltpu.sync_copy(acc, out_hbm.at[w])


def _sc_pool(xf):
    mesh = plsc.VectorSubcoreMesh(core_axis_name="c", subcore_axis_name="s")
    return pl.kernel(
        _sc_pool_body,
        out_type=jax.ShapeDtypeStruct((_NW, _D), jnp.float32),
        mesh=mesh,
        cost_estimate=pl.CostEstimate(
            flops=_BATCH * _TAIL * _D,
            transcendentals=0,
            bytes_accessed=_BATCH * _TAIL * _D * 4,
        ),
        scratch_types=[
            pltpu.VMEM((_G, _D), jnp.float32),
            pltpu.VMEM((_G, _D), jnp.float32),
            pltpu.VMEM((_D,), jnp.float32),
            pltpu.SemaphoreType.DMA,
            pltpu.SemaphoreType.DMA,
        ],
    )(xf)


def _tc_pool_kernel(x_ref, o_ref):
    b = pl.program_id(0)
    s = pl.program_id(1)
    part = jnp.sum(x_ref[...], axis=1)

    @pl.when(s == 0)
    def _():
        o_ref[pl.ds(b, 1), :] = part

    @pl.when(s > 0)
    def _():
        o_ref[pl.ds(b, 1), :] += part


def _router_kernel(tp_ref, sc_ref, w1_ref, b1_ref, g1_ref, be1_ref,
                   w2_ref, b2_ref, g2_ref, be2_ref,
                   w3_ref, b3_ref, o_ref):
    pooled = (tp_ref[...] + jnp.sum(sc_ref[...], axis=1)) * (1.0 / _SEQ)

    def _ln(h, g, bb, eps=1e-5):
        m = jnp.mean(h, axis=-1, keepdims=True)
        v = jnp.mean((h - m) ** 2, axis=-1, keepdims=True)
        return (h - m) / jnp.sqrt(v + eps) * g + bb

    h = jax.lax.dot_general(pooled, w1_ref[...], (((1,), (1,)), ((), ())),
                            preferred_element_type=jnp.float32) + b1_ref[...]
    h = jax.nn.relu(_ln(h, g1_ref[...], be1_ref[...]))
    h = jax.lax.dot_general(h, w2_ref[...], (((1,), (1,)), ((), ())),
                            preferred_element_type=jnp.float32) + b2_ref[...]
    h = jax.nn.relu(_ln(h, g2_ref[...], be2_ref[...]))
    scores = jax.lax.dot_general(h, w3_ref[...], (((1,), (1,)), ((), ())),
                                 preferred_element_type=jnp.float32) + b3_ref[...]

    scaled = scores - jnp.max(scores, axis=-1, keepdims=True)
    e = jnp.exp(scaled - jnp.max(scaled, axis=-1, keepdims=True))
    probs = e / jnp.sum(e, axis=-1, keepdims=True)

    pa = probs[:, :, None]
    pb = probs[:, None, :]
    ii = lax.broadcasted_iota(jnp.int32, (1, _NE, _NE), 1)
    jj = lax.broadcasted_iota(jnp.int32, (1, _NE, _NE), 2)
    beats = (pb > pa) | ((pb == pa) & (jj < ii))
    nbeat = jnp.sum(beats.astype(jnp.int32), axis=-1)
    o_ref[...] = (nbeat < 8).astype(jnp.float32)


def kernel(x, W1, b1, g1, be1, W2, b2, g2, be2, W3, b3):
    xf = x.reshape(_BATCH * _SEQ, _D)

    tc_part = pl.pallas_call(
        _tc_pool_kernel,
        grid=(_BATCH, _HEAD // _SBLK),
        in_specs=[pl.BlockSpec((1, _SBLK, _D), lambda b, s: (b, s, 0))],
        out_specs=pl.BlockSpec((_BATCH, _D), lambda b, s: (0, 0)),
        out_shape=jax.ShapeDtypeStruct((_BATCH, _D), jnp.float32),
    )(x)

    sc_part = _sc_pool(xf).reshape(_BATCH, _WPB, _D)

    full = lambda shape: pl.BlockSpec(shape, lambda: tuple(0 for _ in shape))
    return pl.pallas_call(
        _router_kernel,
        in_specs=[full(tc_part.shape), full(sc_part.shape), full(W1.shape),
                  full(b1.shape), full(g1.shape), full(be1.shape),
                  full(W2.shape), full(b2.shape), full(g2.shape),
                  full(be2.shape), full(W3.shape), full(b3.shape)],
        out_specs=full((_BATCH, _NE)),
        out_shape=jax.ShapeDtypeStruct((_BATCH, _NE), jnp.float32),
    )(tc_part, sc_part, W1, b1, g1, be1, W2, b2, g2, be2, W3, b3)


# speedup vs baseline: 2.2165x; 1.1085x over previous
import jax
import jax.numpy as jnp
from jax import lax
from jax.experimental import pallas as pl
from jax.experimental.pallas import tpu as pltpu

_D = 4096
_SEQ = 8192
_BATCH = 4
_NE = 32
_SBLK = 512
_NS = _SEQ // _SBLK


def _fused_kernel(x_ref, w1_ref, b1_ref, g1_ref, be1_ref,
                  w2_ref, b2_ref, g2_ref, be2_ref,
                  w3_ref, b3_ref, o_ref, acc_ref):
    b = pl.program_id(0)
    s = pl.program_id(1)
    part = jnp.sum(x_ref[...], axis=1)

    @pl.when(s == 0)
    def _():
        acc_ref[pl.ds(b, 1), :] = part

    @pl.when(s > 0)
    def _():
        acc_ref[pl.ds(b, 1), :] += part

    @pl.when((b == _BATCH - 1) & (s == _NS - 1))
    def _():
        pooled = acc_ref[...] * (1.0 / _SEQ)

        def _ln(h, g, bb, eps=1e-5):
            m = jnp.mean(h, axis=-1, keepdims=True)
            v = jnp.mean((h - m) ** 2, axis=-1, keepdims=True)
            return (h - m) / jnp.sqrt(v + eps) * g + bb

        h = lax.dot_general(pooled, w1_ref[...], (((1,), (1,)), ((), ())),
                            preferred_element_type=jnp.float32) + b1_ref[...]
        h = jax.nn.relu(_ln(h, g1_ref[...], be1_ref[...]))
        h = lax.dot_general(h, w2_ref[...], (((1,), (1,)), ((), ())),
                            preferred_element_type=jnp.float32) + b2_ref[...]
        h = jax.nn.relu(_ln(h, g2_ref[...], be2_ref[...]))
        scores = lax.dot_general(h, w3_ref[...], (((1,), (1,)), ((), ())),
                                 preferred_element_type=jnp.float32) + b3_ref[...]

        scaled = scores - jnp.max(scores, axis=-1, keepdims=True)
        e = jnp.exp(scaled - jnp.max(scaled, axis=-1, keepdims=True))
        probs = e / jnp.sum(e, axis=-1, keepdims=True)

        pa = probs[:, :, None]
        pb = probs[:, None, :]
        ii = lax.broadcasted_iota(jnp.int32, (1, _NE, _NE), 1)
        jj = lax.broadcasted_iota(jnp.int32, (1, _NE, _NE), 2)
        beats = (pb > pa) | ((pb == pa) & (jj < ii))
        nbeat = jnp.sum(beats.astype(jnp.int32), axis=-1)
        o_ref[...] = (nbeat < 8).astype(jnp.float32)


def kernel(x, W1, b1, g1, be1, W2, b2, g2, be2, W3, b3):
    const = lambda shape: pl.BlockSpec(shape, lambda b, s: tuple(0 for _ in shape))
    return pl.pallas_call(
        _fused_kernel,
        grid=(_BATCH, _NS),
        in_specs=[pl.BlockSpec((1, _SBLK, _D), lambda b, s: (b, s, 0)),
                  const(W1.shape), const(b1.shape), const(g1.shape),
                  const(be1.shape), const(W2.shape), const(b2.shape),
                  const(g2.shape), const(be2.shape), const(W3.shape),
                  const(b3.shape)],
        out_specs=const((_BATCH, _NE)),
        out_shape=jax.ShapeDtypeStruct((_BATCH, _NE), jnp.float32),
        scratch_shapes=[pltpu.VMEM((_BATCH, _D), jnp.float32)],
    )(x, W1, b1, g1, be1, W2, b2, g2, be2, W3, b3)
